# baseline (speedup 1.0000x reference)
import functools

import jax
import jax.numpy as jnp
import numpy as np
from jax.experimental import pallas as pl
from jax.experimental.pallas import tpu as pltpu

B = 128
V = 100000
K = 5
BR = 8
_TINY = np.float32(np.finfo(np.float32).tiny)
_K0 = np.uint32(0)
_K1 = np.uint32(42)
_K2 = np.uint32(0 ^ 42 ^ 0x1BD11BDA)
_ROTS = ((13, 15, 26, 6), (17, 29, 16, 24))
_KSA = (_K1, _K2, _K0, _K1, _K2)
_KSB = (_K2, _K0, _K1, _K2, _K0)


def _threefry_bits(counts_lo):
    x0 = jnp.full_like(counts_lo, _K0)
    x1 = counts_lo + _K1
    for g in range(5):
        for r in _ROTS[g % 2]:
            x0 = x0 + x1
            x1 = (x1 << np.uint32(r)) | (x1 >> np.uint32(32 - r))
            x1 = x1 ^ x0
        x0 = x0 + _KSA[g]
        x1 = x1 + _KSB[g] + np.uint32(g + 1)
    return x0 ^ x1


def _gumbel_from_bits(bits):
    fb = (bits >> np.uint32(9)) | np.uint32(0x3F800000)
    f = jax.lax.bitcast_convert_type(fb, jnp.float32) - np.float32(1.0)
    u = jnp.maximum(_TINY, f + _TINY)
    return -jnp.log(-jnp.log(u))


def _body(x_ref, samp_ref, ent_ref):
    i = pl.program_id(0)
    x = x_ref[...]

    m = jnp.max(x, axis=1, keepdims=True)
    xm = x - m
    e = jnp.exp(xm)
    s = jnp.sum(e, axis=1, keepdims=True)
    t = jnp.sum(xm * e, axis=1, keepdims=True)
    ent_ref[pl.ds(i * BR, BR), :] = jnp.log(s) - t / s

    viota = jax.lax.broadcasted_iota(jnp.uint32, (BR, V), 1)
    riota = jax.lax.broadcasted_iota(jnp.uint32, (BR, V), 0)
    vidx = jax.lax.broadcasted_iota(jnp.int32, (BR, V), 1)
    row_base = (jnp.uint32(i).astype(jnp.uint32) * np.uint32(BR * V)
                + riota * np.uint32(V) + viota)
    cols = []
    for smp in range(K):
        counts = row_base + np.uint32(smp * B * V)
        y = _gumbel_from_bits(_threefry_bits(counts)) + x
        best = jnp.max(y, axis=1, keepdims=True)
        idx = jnp.min(jnp.where(y == best, vidx, np.int32(2**31 - 1)), axis=1)
        cols.append(idx)
    samp_ref[pl.ds(i * BR, BR), :] = jnp.stack(cols, axis=1)


@jax.jit
def kernel(logits):
    samp_t, ent = pl.pallas_call(
        _body,
        grid=(B // BR,),
        in_specs=[pl.BlockSpec((BR, V), lambda i: (i, 0))],
        out_specs=[
            pl.BlockSpec((B, K), lambda i: (0, 0)),
            pl.BlockSpec((B, 1), lambda i: (0, 0)),
        ],
        out_shape=[
            jax.ShapeDtypeStruct((B, K), jnp.int32),
            jax.ShapeDtypeStruct((B, 1), jnp.float32),
        ],
        compiler_params=pltpu.CompilerParams(
            dimension_semantics=("arbitrary",),
        ),
    )(logits)
    return samp_t.T, logits, ent[:, 0]
